# baseline (speedup 1.0000x reference)
import jax
import jax.numpy as jnp
from jax.experimental import pallas as pl

_B, _C, _H, _W = 16, 384, 56, 56
_PS = 3
_STRIDE = 26
_NP = 9


_BB = 16


def _crop_kernel(xt_ref, out_ref):
    nb = pl.program_id(1)
    mb = pl.program_id(2)
    roff = 2 * nb
    woff = 2 * mb
    for pr in range(_PS):
        for pc in range(_PS):
            out_ref[:, _PS * nb + mb, _PS * pr + pc, :] = (
                xt_ref[:, roff + pr, woff + pc, :]
            )


@jax.jit
def kernel(x):
    xt = jnp.transpose(x, (0, 2, 3, 1))
    out5 = pl.pallas_call(
        _crop_kernel,
        grid=(_B // _BB, _PS, _PS),
        in_specs=[
            pl.BlockSpec(
                (_BB, 8, 8, _C),
                lambda b, nb, mb: (b, 3 * nb, 3 * mb, 0),
            ),
        ],
        out_specs=pl.BlockSpec(
            (_BB, _NP, _PS * _PS, _C),
            lambda b, nb, mb: (b, 0, 0, 0),
        ),
        out_shape=jax.ShapeDtypeStruct((_B, _NP, _PS * _PS, _C), jnp.float32),
    )(xt)
    perm = jnp.asarray(
        [(k % _NP) * _C + k // _NP for k in range(_C * _NP)], jnp.int32)
    flat = out5.reshape(_B, _NP, _PS * _PS * _C)
    return jnp.take(flat, perm, axis=2)


# speedup vs baseline: 2.0696x; 2.0696x over previous
import jax
import jax.numpy as jnp
from jax.experimental import pallas as pl

_B, _C, _H, _W = 16, 384, 56, 56
_PS = 3
_NP = 9
_BB = 16


def _crop_kernel(xt_ref, out_ref):
    nb = pl.program_id(1)
    mb = pl.program_id(2)
    roff = 2 * nb
    woff = 2 * mb
    for pr in range(_PS):
        for pc in range(_PS):
            out_ref[:, _PS * nb + mb, _PS * pr + pc, :] = (
                xt_ref[:, roff + pr, woff + pc, :]
            )


@jax.jit
def kernel(x):
    xt = jnp.transpose(x, (0, 2, 3, 1))
    out5 = pl.pallas_call(
        _crop_kernel,
        grid=(_B // _BB, _PS, _PS),
        in_specs=[
            pl.BlockSpec(
                (_BB, 8, 8, _C),
                lambda b, nb, mb: (b, 3 * nb, 3 * mb, 0),
            ),
        ],
        out_specs=pl.BlockSpec(
            (_BB, _NP, _PS * _PS, _C),
            lambda b, nb, mb: (b, 0, 0, 0),
        ),
        out_shape=jax.ShapeDtypeStruct((_B, _NP, _PS * _PS, _C), jnp.float32),
    )(xt)
    return jnp.transpose(out5, (0, 1, 3, 2)).reshape(_B, _NP, _C * _PS * _PS)
